# baseline (speedup 1.0000x reference)
import functools

import jax
import jax.numpy as jnp
import numpy as np
from jax import lax
from jax.experimental import pallas as pl
from jax.experimental.pallas import tpu as pltpu
from jax.experimental.pallas import tpu_sc as plsc

B = 16
Q = 300
D = 256
NH = 8
NL = 3
NP = 4
DH = D // NH
BQ = B * Q
NS = NH * NL * NP
NSAMP = NS * 4
LVL_W = (80, 40, 20)
LVL_H = (80, 40, 20)
LVL_BASE = (0, 6400, 8000)
S = 8400


_VROWS = 2400


def _vproj_body(x_ref, w_ref, b_ref, o_ref):
    o_ref[:] = (
        jnp.dot(x_ref[:], w_ref[:], preferred_element_type=jnp.float32)
        + b_ref[:]
    )


def _vproj2_body(x_ref, w_ref, b_ref, o_ref):
    r = (jnp.dot(x_ref[:], w_ref[:], preferred_element_type=jnp.float32)
         + b_ref[:])
    o_ref[:] = r.astype(jnp.bfloat16)


def _value_projection(ehs2, Wv, bv):
    m = ehs2.shape[0]
    grid = m // _VROWS
    return pl.pallas_call(
        _vproj2_body,
        grid=(grid,),
        in_specs=[
            pl.BlockSpec((_VROWS, D), lambda i: (i, 0)),
            pl.BlockSpec((D, D), lambda i: (0, 0)),
            pl.BlockSpec((1, D), lambda i: (0, 0)),
        ],
        out_specs=pl.BlockSpec((_VROWS, D), lambda i: (i, 0)),
        out_shape=jax.ShapeDtypeStruct((m, D), jnp.bfloat16),
    )(ehs2, Wv, bv.reshape(1, D))


_PROWS = 600


def _prep_body(hs_ref, rp_ref, wx_ref, wy_ref, bx_ref, by_ref, wa_ref, ba_ref,
               aw_ref, idx_ref, w_ref):
    f32 = jnp.float32
    hi = jax.lax.Precision.HIGHEST
    hs = hs_ref[:]
    offx = jnp.dot(hs, wx_ref[:], preferred_element_type=f32,
                   precision=hi) + bx_ref[:]
    offy = jnp.dot(hs, wy_ref[:], preferred_element_type=f32,
                   precision=hi) + by_ref[:]
    logits = jnp.dot(hs, wa_ref[:], preferred_element_type=f32,
                     precision=hi) + ba_ref[:]

    m = jnp.max(logits, axis=1, keepdims=True)
    e = jnp.exp(logits - m)
    gr = lax.broadcasted_iota(jnp.int32, (NS, NS), 0) // (NL * NP)
    gc = lax.broadcasted_iota(jnp.int32, (NS, NS), 1) // (NL * NP)
    group = (gr == gc).astype(f32)
    denom = jnp.dot(e, group, preferred_element_type=f32, precision=hi)
    aw = e / denom
    aw_ref[:] = aw

    col = lax.broadcasted_iota(jnp.int32, (_PROWS, NS), 1)
    lvl = (col // NP) % NL
    h = col // (NL * NP)
    wn = jnp.where(lvl == 0, float(LVL_W[0]),
                   jnp.where(lvl == 1, float(LVL_W[1]), float(LVL_W[2])))
    hn = jnp.where(lvl == 0, float(LVL_H[0]),
                   jnp.where(lvl == 1, float(LVL_H[1]), float(LVL_H[2])))
    base = jnp.where(lvl == 0, LVL_BASE[0],
                     jnp.where(lvl == 1, LVL_BASE[1], LVL_BASE[2]))
    wi = jnp.where(lvl == 0, LVL_W[0],
                   jnp.where(lvl == 1, LVL_W[1], LVL_W[2]))

    rp = rp_ref[:]
    rpx_h = jnp.concatenate(
        [jnp.broadcast_to(rp[:, 2 * l:2 * l + 1], (_PROWS, NP))
         for l in range(NL)], axis=1)
    rpy_h = jnp.concatenate(
        [jnp.broadcast_to(rp[:, 2 * l + 1:2 * l + 2], (_PROWS, NP))
         for l in range(NL)], axis=1)
    rpx = jnp.concatenate([rpx_h] * NH, axis=1)
    rpy = jnp.concatenate([rpy_h] * NH, axis=1)

    gx = rpx * wn + offx - 0.5
    gy = rpy * hn + offy - 0.5
    x0 = jnp.floor(gx)
    y0 = jnp.floor(gy)
    wx1 = gx - x0
    wy1 = gy - y0
    wx0 = 1.0 - wx1
    wy0 = 1.0 - wy1

    rowg = pl.program_id(0) * _PROWS + lax.broadcasted_iota(
        jnp.int32, (_PROWS, NS), 0)
    bb = rowg // Q
    row_base = bb * (S * NH) + base * NH + h

    idx_parts = []
    w_parts = []
    for cx, cy, wxc, wyc in ((0., 0., wx0, wy0), (1., 0., wx1, wy0),
                             (0., 1., wx0, wy1), (1., 1., wx1, wy1)):
        xi = x0 + cx
        yi = y0 + cy
        valid = ((xi >= 0.0) & (xi <= wn - 1.0)
                 & (yi >= 0.0) & (yi <= hn - 1.0)).astype(f32)
        xc = jnp.clip(xi, 0.0, wn - 1.0).astype(jnp.int32)
        yc = jnp.clip(yi, 0.0, hn - 1.0).astype(jnp.int32)
        idx_parts.append(row_base + (yc * wi + xc) * NH)
        w_parts.append(wxc * wyc * valid * aw)
    idx_ref[:] = jnp.concatenate(idx_parts, axis=1)
    w_ref[:] = jnp.concatenate(w_parts, axis=1)


def _sampling_prep(hs2, rp2, Woff, boff, Wattn, battn):
    Woff_x = Woff[:, 0::2]
    Woff_y = Woff[:, 1::2]
    boff_x = boff[0::2].reshape(1, NS)
    boff_y = boff[1::2].reshape(1, NS)
    grid = BQ // _PROWS
    return pl.pallas_call(
        _prep_body,
        grid=(grid,),
        in_specs=[
            pl.BlockSpec((_PROWS, D), lambda i: (i, 0)),
            pl.BlockSpec((_PROWS, 2 * NL), lambda i: (i, 0)),
            pl.BlockSpec((D, NS), lambda i: (0, 0)),
            pl.BlockSpec((D, NS), lambda i: (0, 0)),
            pl.BlockSpec((1, NS), lambda i: (0, 0)),
            pl.BlockSpec((1, NS), lambda i: (0, 0)),
            pl.BlockSpec((D, NS), lambda i: (0, 0)),
            pl.BlockSpec((1, NS), lambda i: (0, 0)),
        ],
        out_specs=[
            pl.BlockSpec((_PROWS, NS), lambda i: (i, 0)),
            pl.BlockSpec((_PROWS, NSAMP), lambda i: (i, 0)),
            pl.BlockSpec((_PROWS, NSAMP), lambda i: (i, 0)),
        ],
        out_shape=[
            jax.ShapeDtypeStruct((BQ, NS), jnp.float32),
            jax.ShapeDtypeStruct((BQ, NSAMP), jnp.int32),
            jax.ShapeDtypeStruct((BQ, NSAMP), jnp.float32),
        ],
    )(hs2, rp2, Woff_x, Woff_y, boff_x, boff_y, Wattn,
      battn.reshape(1, NS))


_NW = 32
_RPW = BQ // _NW
_NCHK = 3
_CHK = NSAMP // _NCHK


_NBUF = 10
_GD = 5
_FD = 9


def _sc_body(idx_hbm, w_hbm, t_hbm, out_hbm, idx_v, w_v, rows_v, out_acc,
             fsem, gsem):
    wid = lax.axis_index("s") * 2 + lax.axis_index("c")
    base = wid * _RPW
    last = _RPW - 1

    def slot_of(g):
        return lax.rem(g, _NBUF) if not isinstance(g, int) else g % _NBUF

    def fetch(g):
        slot = slot_of(g)
        r = base + jnp.minimum(g, last)
        pltpu.async_copy(idx_hbm.at[r], idx_v.at[slot], fsem.at[slot])
        pltpu.async_copy(w_hbm.at[r], w_v.at[slot, pl.ds(0, NSAMP)],
                         fsem.at[slot])

    def wait_fetch(g):
        slot = slot_of(g)
        pltpu.make_async_copy(idx_hbm.at[0], idx_v.at[slot],
                              fsem.at[slot]).wait()
        pltpu.make_async_copy(w_hbm.at[0], w_v.at[slot, pl.ds(0, NSAMP)],
                              fsem.at[slot]).wait()

    def gather(g):
        slot = slot_of(g)
        for i in range(_NCHK):
            pltpu.async_copy(t_hbm.at[idx_v.at[slot, i]], rows_v.at[slot, i],
                             gsem.at[slot])

    def wait_gather(g):
        slot = slot_of(g)
        for i in range(_NCHK):
            pltpu.make_async_copy(t_hbm.at[idx_v.at[slot, i]],
                                  rows_v.at[slot, i], gsem.at[slot]).wait()

    def compute(g):
        slot = slot_of(g)
        for h in range(NH):
            a0 = jnp.zeros((16,), jnp.float32)
            a1 = jnp.zeros((16,), jnp.float32)
            for c in range(4):
                start = c * NS + h * (NL * NP)
                wvec = w_v[slot, pl.ds(start, 16)]
                for k in range(NL * NP):
                    j = start + k
                    wj = wvec[k]
                    row = rows_v[slot, j // _CHK, j % _CHK, :]
                    ev, od = plsc.unpack(row,
                                         format=plsc.PackFormat.INTERLEAVED)
                    a0 = a0 + wj * ev
                    a1 = a1 + wj * od
            out_acc[g, pl.ds(h * DH, 16)] = a0
            out_acc[g, pl.ds(h * DH + 16, 16)] = a1

    for g in range(_FD):
        fetch(g)
    for g in range(_GD):
        wait_fetch(g)
        gather(g)

    def row_body(r, carry):
        wait_fetch(r + _GD)
        gather(r + _GD)
        fetch(r + _FD)
        wait_gather(r)
        compute(r)
        return carry

    lax.fori_loop(0, _RPW, row_body, 0)
    for d in range(_GD):
        wait_gather(_RPW + d)
    for d in range(_GD, _FD):
        wait_fetch(_RPW + d)
    pltpu.sync_copy(out_acc, out_hbm.at[pl.ds(base, _RPW)])


def _sc_sample(idx3, w2, table):
    mesh = plsc.VectorSubcoreMesh(core_axis_name="c", subcore_axis_name="s")
    fn = pl.kernel(
        _sc_body,
        mesh=mesh,
        out_type=jax.ShapeDtypeStruct((BQ, D), jnp.float32),
        scratch_types=[
            pltpu.VMEM((_NBUF, _NCHK, _CHK), jnp.int32),
            pltpu.VMEM((_NBUF, NSAMP + 16), jnp.float32),
            pltpu.VMEM((_NBUF, _NCHK, _CHK, DH), jnp.bfloat16),
            pltpu.VMEM((_RPW, D), jnp.float32),
            pltpu.SemaphoreType.DMA((_NBUF,)),
            pltpu.SemaphoreType.DMA((_NBUF,)),
        ],
        compiler_params=pltpu.CompilerParams(use_tc_tiling_on_sc=False,
                                             needs_layout_passes=False),
    )
    return fn(idx3, w2, table)


_OROWS = 1200


def _out_projection(x2, Wout, bout):
    grid = BQ // _OROWS
    return pl.pallas_call(
        _vproj_body,
        grid=(grid,),
        in_specs=[
            pl.BlockSpec((_OROWS, D), lambda i: (i, 0)),
            pl.BlockSpec((D, D), lambda i: (0, 0)),
            pl.BlockSpec((1, D), lambda i: (0, 0)),
        ],
        out_specs=pl.BlockSpec((_OROWS, D), lambda i: (i, 0)),
        out_shape=jax.ShapeDtypeStruct((BQ, D), jnp.float32),
    )(x2, Wout, bout.reshape(1, D))


_PERM = np.concatenate([
    h * DH + np.concatenate([np.arange(0, DH, 2), np.arange(1, DH, 2)])
    for h in range(NH)
])


def kernel(hidden_states, encoder_hidden_states, reference_points, Wv, bv,
           Woff, boff, Wattn, battn, Wout, bout):
    hs2 = hidden_states.reshape(BQ, D)
    rp2 = reference_points.reshape(BQ, 2 * NL)
    aw, idx, w = _sampling_prep(hs2, rp2, Woff, boff, Wattn, battn)

    ehs2 = encoder_hidden_states.reshape(B * S, D)
    table = _value_projection(ehs2, Wv, bv).reshape(B * S * NH, DH)

    idx3 = idx.reshape(BQ, _NCHK, _CHK)
    sampled = _sc_sample(idx3, w, table)

    Wout_p = Wout[_PERM, :]
    out = _out_projection(sampled, Wout_p, bout).reshape(B, Q, D)
    return out, aw.reshape(B, Q, NH, NL * NP)


# speedup vs baseline: 1.1714x; 1.1714x over previous
import functools

import jax
import jax.numpy as jnp
import numpy as np
from jax import lax
from jax.experimental import pallas as pl
from jax.experimental.pallas import tpu as pltpu
from jax.experimental.pallas import tpu_sc as plsc

B = 16
Q = 300
D = 256
NH = 8
NL = 3
NP = 4
DH = D // NH
BQ = B * Q
NS = NH * NL * NP
NSAMP = NS * 4
LVL_W = (80, 40, 20)
LVL_H = (80, 40, 20)
LVL_BASE = (0, 6400, 8000)
S = 8400


_VROWS = 2400


def _vproj_body(x_ref, w_ref, b_ref, o_ref):
    o_ref[:] = (
        jnp.dot(x_ref[:], w_ref[:], preferred_element_type=jnp.float32)
        + b_ref[:]
    )


def _vproj2_body(x_ref, w_ref, b_ref, o_ref):
    r = (jnp.dot(x_ref[:], w_ref[:], preferred_element_type=jnp.float32)
         + b_ref[:])
    o_ref[:] = r.astype(jnp.bfloat16)


def _value_projection(ehs2, Wv, bv):
    m = ehs2.shape[0]
    grid = m // _VROWS
    return pl.pallas_call(
        _vproj2_body,
        grid=(grid,),
        in_specs=[
            pl.BlockSpec((_VROWS, D), lambda i: (i, 0)),
            pl.BlockSpec((D, D), lambda i: (0, 0)),
            pl.BlockSpec((1, D), lambda i: (0, 0)),
        ],
        out_specs=pl.BlockSpec((_VROWS, D), lambda i: (i, 0)),
        out_shape=jax.ShapeDtypeStruct((m, D), jnp.bfloat16),
    )(ehs2, Wv, bv.reshape(1, D))


_PROWS = 600


def _prep_body(hs_ref, rp_ref, wx_ref, wy_ref, bx_ref, by_ref, wa_ref, ba_ref,
               aw_ref, idx_ref, w_ref):
    f32 = jnp.float32
    hi = jax.lax.Precision.HIGHEST
    hs = hs_ref[:]
    offx = jnp.dot(hs, wx_ref[:], preferred_element_type=f32,
                   precision=hi) + bx_ref[:]
    offy = jnp.dot(hs, wy_ref[:], preferred_element_type=f32,
                   precision=hi) + by_ref[:]
    logits = jnp.dot(hs, wa_ref[:], preferred_element_type=f32,
                     precision=hi) + ba_ref[:]

    m = jnp.max(logits, axis=1, keepdims=True)
    e = jnp.exp(logits - m)
    gr = lax.broadcasted_iota(jnp.int32, (NS, NS), 0) // (NL * NP)
    gc = lax.broadcasted_iota(jnp.int32, (NS, NS), 1) // (NL * NP)
    group = (gr == gc).astype(f32)
    denom = jnp.dot(e, group, preferred_element_type=f32, precision=hi)
    aw = e / denom
    aw_ref[:] = aw

    col = lax.broadcasted_iota(jnp.int32, (_PROWS, NS), 1)
    lvl = (col // NP) % NL
    h = col // (NL * NP)
    wn = jnp.where(lvl == 0, float(LVL_W[0]),
                   jnp.where(lvl == 1, float(LVL_W[1]), float(LVL_W[2])))
    hn = jnp.where(lvl == 0, float(LVL_H[0]),
                   jnp.where(lvl == 1, float(LVL_H[1]), float(LVL_H[2])))
    base = jnp.where(lvl == 0, LVL_BASE[0],
                     jnp.where(lvl == 1, LVL_BASE[1], LVL_BASE[2]))
    wi = jnp.where(lvl == 0, LVL_W[0],
                   jnp.where(lvl == 1, LVL_W[1], LVL_W[2]))

    rp = rp_ref[:]
    rpx_h = jnp.concatenate(
        [jnp.broadcast_to(rp[:, 2 * l:2 * l + 1], (_PROWS, NP))
         for l in range(NL)], axis=1)
    rpy_h = jnp.concatenate(
        [jnp.broadcast_to(rp[:, 2 * l + 1:2 * l + 2], (_PROWS, NP))
         for l in range(NL)], axis=1)
    rpx = jnp.concatenate([rpx_h] * NH, axis=1)
    rpy = jnp.concatenate([rpy_h] * NH, axis=1)

    gx = rpx * wn + offx - 0.5
    gy = rpy * hn + offy - 0.5
    x0 = jnp.floor(gx)
    y0 = jnp.floor(gy)
    wx1 = gx - x0
    wy1 = gy - y0
    wx0 = 1.0 - wx1
    wy0 = 1.0 - wy1

    rowg = pl.program_id(0) * _PROWS + lax.broadcasted_iota(
        jnp.int32, (_PROWS, NS), 0)
    bb = rowg // Q
    row_base = bb * (S * NH) + base * NH + h

    idx_parts = []
    w_parts = []
    for cx, cy, wxc, wyc in ((0., 0., wx0, wy0), (1., 0., wx1, wy0),
                             (0., 1., wx0, wy1), (1., 1., wx1, wy1)):
        xi = x0 + cx
        yi = y0 + cy
        valid = ((xi >= 0.0) & (xi <= wn - 1.0)
                 & (yi >= 0.0) & (yi <= hn - 1.0)).astype(f32)
        xc = jnp.clip(xi, 0.0, wn - 1.0).astype(jnp.int32)
        yc = jnp.clip(yi, 0.0, hn - 1.0).astype(jnp.int32)
        idx_parts.append(row_base + (yc * wi + xc) * NH)
        w_parts.append(wxc * wyc * valid * aw)
    idx_ref[:] = jnp.concatenate(idx_parts, axis=1).reshape(
        _PROWS * NSAMP // 128, 128)
    zpad = jnp.zeros((_PROWS, 128 - NS), jnp.float32)
    w_padded = []
    for p in w_parts:
        w_padded.append(p)
        w_padded.append(zpad)
    w_ref[:] = jnp.concatenate(w_padded, axis=1).reshape(
        _PROWS * 4, 128)


def _sampling_prep(hs2, rp2, Woff, boff, Wattn, battn):
    Woff_x = Woff[:, 0::2]
    Woff_y = Woff[:, 1::2]
    boff_x = boff[0::2].reshape(1, NS)
    boff_y = boff[1::2].reshape(1, NS)
    grid = BQ // _PROWS
    return pl.pallas_call(
        _prep_body,
        grid=(grid,),
        in_specs=[
            pl.BlockSpec((_PROWS, D), lambda i: (i, 0)),
            pl.BlockSpec((_PROWS, 2 * NL), lambda i: (i, 0)),
            pl.BlockSpec((D, NS), lambda i: (0, 0)),
            pl.BlockSpec((D, NS), lambda i: (0, 0)),
            pl.BlockSpec((1, NS), lambda i: (0, 0)),
            pl.BlockSpec((1, NS), lambda i: (0, 0)),
            pl.BlockSpec((D, NS), lambda i: (0, 0)),
            pl.BlockSpec((1, NS), lambda i: (0, 0)),
        ],
        out_specs=[
            pl.BlockSpec((_PROWS, NS), lambda i: (i, 0)),
            pl.BlockSpec((_PROWS * NSAMP // 128, 128), lambda i: (i, 0)),
            pl.BlockSpec((_PROWS * 4, 128), lambda i: (i, 0)),
        ],
        out_shape=[
            jax.ShapeDtypeStruct((BQ, NS), jnp.float32),
            jax.ShapeDtypeStruct((BQ * NSAMP // 128, 128), jnp.int32),
            jax.ShapeDtypeStruct((BQ * 4, 128), jnp.float32),
        ],
    )(hs2, rp2, Woff_x, Woff_y, boff_x, boff_y, Wattn,
      battn.reshape(1, NS))


_NW = 32
_RPW = BQ // _NW
_NCHK = 3
_CHK = NSAMP // _NCHK


_NBUF = 10
_GD = 5
_FD = 9


def _sc_body(idx_hbm, w_hbm, t_hbm, out_hbm, idx_v, w_v, rows_v, out_acc,
             fsem, gsem):
    wid = lax.axis_index("s") * 2 + lax.axis_index("c")
    base = wid * _RPW
    last = _RPW - 1

    def slot_of(g):
        return lax.rem(g, _NBUF) if not isinstance(g, int) else g % _NBUF

    def fetch(g):
        slot = slot_of(g)
        r = base + jnp.minimum(g, last)
        pltpu.async_copy(idx_hbm.at[pl.ds(_NCHK * r, _NCHK)], idx_v.at[slot],
                         fsem.at[slot])
        pltpu.async_copy(w_hbm.at[pl.ds(4 * r, 4)], w_v.at[slot],
                         fsem.at[slot])

    def wait_fetch(g):
        slot = slot_of(g)
        pltpu.make_async_copy(idx_hbm.at[pl.ds(0, _NCHK)], idx_v.at[slot],
                              fsem.at[slot]).wait()
        pltpu.make_async_copy(w_hbm.at[pl.ds(0, 4)], w_v.at[slot],
                              fsem.at[slot]).wait()

    def gather(g):
        slot = slot_of(g)
        for i in range(_NCHK):
            pltpu.async_copy(t_hbm.at[idx_v.at[slot, i]], rows_v.at[slot, i],
                             gsem.at[slot])

    def wait_gather(g):
        slot = slot_of(g)
        for i in range(_NCHK):
            pltpu.make_async_copy(t_hbm.at[idx_v.at[slot, i]],
                                  rows_v.at[slot, i], gsem.at[slot]).wait()

    def compute(g):
        slot = slot_of(g)
        for h in range(NH):
            a0 = jnp.zeros((16,), jnp.float32)
            a1 = jnp.zeros((16,), jnp.float32)
            for c in range(4):
                start = c * NS + h * (NL * NP)
                wvec = w_v[slot, c, pl.ds(h * (NL * NP), 16)]
                for k in range(NL * NP):
                    j = start + k
                    wj = wvec[k]
                    row = rows_v[slot, j // _CHK, j % _CHK, :]
                    ev, od = plsc.unpack(row,
                                         format=plsc.PackFormat.INTERLEAVED)
                    a0 = a0 + wj * ev
                    a1 = a1 + wj * od
            out_acc[g, pl.ds(h * DH, 16)] = a0
            out_acc[g, pl.ds(h * DH + 16, 16)] = a1

    for g in range(_FD):
        fetch(g)
    for g in range(_GD):
        wait_fetch(g)
        gather(g)

    def row_body(r, carry):
        wait_fetch(r + _GD)
        gather(r + _GD)
        fetch(r + _FD)
        wait_gather(r)
        compute(r)
        return carry

    lax.fori_loop(0, _RPW, row_body, 0)
    for d in range(_GD):
        wait_gather(_RPW + d)
    for d in range(_GD, _FD):
        wait_fetch(_RPW + d)
    pltpu.sync_copy(out_acc, out_hbm.at[pl.ds(base, _RPW)])


def _sc_sample(idx3, w2, table):
    mesh = plsc.VectorSubcoreMesh(core_axis_name="c", subcore_axis_name="s")
    fn = pl.kernel(
        _sc_body,
        mesh=mesh,
        out_type=jax.ShapeDtypeStruct((BQ, D), jnp.float32),
        scratch_types=[
            pltpu.VMEM((_NBUF, _NCHK, _CHK), jnp.int32),
            pltpu.VMEM((_NBUF, 4, 128), jnp.float32),
            pltpu.VMEM((_NBUF, _NCHK, _CHK, DH), jnp.bfloat16),
            pltpu.VMEM((_RPW, D), jnp.float32),
            pltpu.SemaphoreType.DMA((_NBUF,)),
            pltpu.SemaphoreType.DMA((_NBUF,)),
        ],
        compiler_params=pltpu.CompilerParams(use_tc_tiling_on_sc=False,
                                             needs_layout_passes=False),
    )
    return fn(idx3, w2, table)


_OROWS = 1200


def _out_projection(x2, Wout, bout):
    grid = BQ // _OROWS
    return pl.pallas_call(
        _vproj_body,
        grid=(grid,),
        in_specs=[
            pl.BlockSpec((_OROWS, D), lambda i: (i, 0)),
            pl.BlockSpec((D, D), lambda i: (0, 0)),
            pl.BlockSpec((1, D), lambda i: (0, 0)),
        ],
        out_specs=pl.BlockSpec((_OROWS, D), lambda i: (i, 0)),
        out_shape=jax.ShapeDtypeStruct((BQ, D), jnp.float32),
    )(x2, Wout, bout.reshape(1, D))


_PERM = np.concatenate([
    h * DH + np.concatenate([np.arange(0, DH, 2), np.arange(1, DH, 2)])
    for h in range(NH)
])


def kernel(hidden_states, encoder_hidden_states, reference_points, Wv, bv,
           Woff, boff, Wattn, battn, Wout, bout):
    hs2 = hidden_states.reshape(BQ, D)
    rp2 = reference_points.reshape(BQ, 2 * NL)
    aw, idx, w = _sampling_prep(hs2, rp2, Woff, boff, Wattn, battn)

    ehs2 = encoder_hidden_states.reshape(B * S, D)
    table = _value_projection(ehs2, Wv, bv).reshape(B * S * NH, DH)

    sampled = _sc_sample(idx, w, table)

    Wout_p = Wout[_PERM, :]
    out = _out_projection(sampled, Wout_p, bout).reshape(B, Q, D)
    return out, aw.reshape(B, Q, NH, NL * NP)
